# Initial kernel scaffold; baseline (speedup 1.0000x reference)
import jax
import jax.numpy as jnp
from jax.experimental import pallas as pl


def kernel(x, edge_index, edge_attr, W, b, root_emb, bond_table):
    raise NotImplementedError("write your pallas kernel here")


# speedup vs baseline: 1.4004x; 1.4004x over previous
import functools

import jax
import jax.numpy as jnp
from jax import lax
from jax.experimental import pallas as pl
from jax.experimental.pallas import tpu as pltpu
from jax.experimental.pallas import tpu_sc as plsc

N = 10000
E = 160000
D = 256
HALF = 128
NPAD = 10240
NROWS = NPAD // 128
PAD_IDX = 10200

EA = E // 32
EA_PAD = 5008
_A_CHUNKS = EA_PAD // 16

_mesh = plsc.VectorSubcoreMesh(core_axis_name="c", subcore_axis_name="s")
_sc_params = pltpu.CompilerParams(needs_layout_passes=False,
                                  use_tc_tiling_on_sc=False)


_SEG = NPAD // 16


@functools.partial(
    pl.kernel,
    out_type=jax.ShapeDtypeStruct((2, NPAD), jnp.float32),
    mesh=_mesh,
    scratch_types=[
        pltpu.VMEM((_A_CHUNKS, 16), jnp.int32),
        pltpu.VMEM((NPAD,), jnp.float32),
        pltpu.VMEM((16, _SEG), jnp.float32),
        pltpu.VMEM((_SEG,), jnp.float32),
        pltpu.VMEM_SHARED((16, NPAD), jnp.float32),
    ],
    compiler_params=_sc_params,
)
def _deg_kernel(rows_hbm, out_hbm, rvm, hist, stage, seg, shared):
    c = lax.axis_index("c")
    s = lax.axis_index("s")
    tid = c * 16 + s
    zeros16 = jnp.zeros((16,), jnp.float32)

    def zrow(j, _):
        hist[pl.ds(j * 16, 16)] = zeros16
        return 0
    lax.fori_loop(0, NPAD // 16, zrow, 0)

    pltpu.sync_copy(rows_hbm.at[tid], rvm)

    ones = jnp.full((16,), 1.0, jnp.float32)

    def chunk(j, _):
        plsc.addupdate_scatter(hist, [rvm[j]], ones)
        return 0
    lax.fori_loop(0, _A_CHUNKS, chunk, 0)

    pltpu.sync_copy(hist, shared.at[s])
    plsc.subcore_barrier()
    pltpu.sync_copy(shared.at[pl.ds(0, 16), pl.ds(s * _SEG, _SEG)], stage)

    def red(k, _):
        v = stage[0, pl.ds(k * 16, 16)]
        for r in range(1, 16):
            v = v + stage[r, pl.ds(k * 16, 16)]
        seg[pl.ds(k * 16, 16)] = v
        return 0
    lax.fori_loop(0, _SEG // 16, red, 0)

    pltpu.sync_copy(seg, out_hbm.at[c, pl.ds(s * _SEG, _SEG)])


BM = 1024
_GRID = (NPAD // BM,)


def _mm_body(x_ref, w_ref, b_ref, root_ref, degp_ref,
             h2_ref, base2_ref, dis_ref):
    h = jnp.dot(x_ref[...], w_ref[...],
                preferred_element_type=jnp.float32) + b_ref[...]
    deg = degp_ref[0] + degp_ref[1] + 1.0
    dis = lax.rsqrt(deg)
    inv = dis * dis
    base = jnp.maximum(h + root_ref[...], 0.0) * inv
    h2_ref[0] = h[:, :HALF]
    h2_ref[1] = h[:, HALF:]
    base2_ref[0] = base[:, :HALF]
    base2_ref[1] = base[:, HALF:]
    dis_ref[...] = dis


_mm_call = pl.pallas_call(
    _mm_body,
    grid=_GRID,
    in_specs=[
        pl.BlockSpec((BM, D), lambda i: (i, 0)),
        pl.BlockSpec((D, D), lambda i: (0, 0)),
        pl.BlockSpec((1, D), lambda i: (0, 0)),
        pl.BlockSpec((1, D), lambda i: (0, 0)),
        pl.BlockSpec((2, BM, 1), lambda i: (0, i, 0)),
    ],
    out_specs=[
        pl.BlockSpec((2, BM, HALF), lambda i: (0, i, 0)),
        pl.BlockSpec((2, BM, HALF), lambda i: (0, i, 0)),
        pl.BlockSpec((BM, 1), lambda i: (i, 0)),
    ],
    out_shape=[
        jax.ShapeDtypeStruct((2, N, HALF), jnp.float32),
        jax.ShapeDtypeStruct((2, N, HALF), jnp.float32),
        jax.ShapeDtypeStruct((NPAD, 1), jnp.float32),
    ],
)


EC = E // 16
_C_CHUNKS = EC // 16
_RPT = 624


@functools.partial(
    pl.kernel,
    out_type=jax.ShapeDtypeStruct((2, N, HALF), jnp.float32),
    mesh=_mesh,
    scratch_types=[
        pltpu.VMEM((_C_CHUNKS, 16), jnp.int32),
        pltpu.VMEM((_C_CHUNKS, 16), jnp.int32),
        pltpu.VMEM((_C_CHUNKS, 16), jnp.int32),
        pltpu.VMEM((NPAD,), jnp.float32),
        pltpu.VMEM((8, HALF), jnp.float32),
        pltpu.VMEM((16, HALF), jnp.float32),
        pltpu.VMEM((16, HALF), jnp.float32),
        pltpu.VMEM_SHARED((N, HALF), jnp.float32),
        pltpu.SemaphoreType.DMA,
    ],
    compiler_params=_sc_params,
)
def _msg_kernel(row_hbm, col_hbm, attr_hbm, dis_hbm, bond_hbm, h_hbm,
                base_hbm, out_hbm, rvm, cvm, avm, disv, bondv, hbuf, msg,
                accum, sem):
    c = lax.axis_index("c")
    s = lax.axis_index("s")
    lanes = lax.iota(jnp.int32, 16)

    pltpu.sync_copy(row_hbm.at[s], rvm)
    pltpu.sync_copy(col_hbm.at[s], cvm)
    pltpu.sync_copy(attr_hbm.at[s], avm)
    pltpu.sync_copy(dis_hbm, disv)
    pltpu.sync_copy(bond_hbm.at[c], bondv)
    pltpu.sync_copy(base_hbm.at[c, pl.ds(s * _RPT, _RPT)],
                    accum.at[pl.ds(s * _RPT, _RPT)])

    @pl.when(s == 15)
    def _():
        pltpu.sync_copy(base_hbm.at[c, pl.ds(16 * _RPT, N - 16 * _RPT)],
                        accum.at[pl.ds(16 * _RPT, N - 16 * _RPT)])
    plsc.subcore_barrier()

    def chunk(j, _):
        pltpu.async_copy(h_hbm.at[c].at[rvm.at[j]], hbuf, sem).wait()
        rvec = rvm[j]
        cvec = cvm[j]
        avec = avm[j]
        norm = (plsc.load_gather(disv, [rvec]) *
                plsc.load_gather(disv, [cvec]))

        def dbody(d, _):
            dv = jnp.full((16,), d, jnp.int32)
            bv = plsc.load_gather(bondv, [avec, dv])
            hv = plsc.load_gather(hbuf, [lanes, dv])
            v = jnp.maximum(hv + bv, 0.0) * norm
            plsc.store_scatter(msg, [lanes, dv], v)
            return 0
        lax.fori_loop(0, HALF, dbody, 0)

        pltpu.sync_copy(msg, accum.at[cvm.at[j]], add=True)
        return 0
    lax.fori_loop(0, _C_CHUNKS, chunk, 0)

    plsc.subcore_barrier()
    pltpu.sync_copy(accum.at[pl.ds(s * _RPT, _RPT)],
                    out_hbm.at[c, pl.ds(s * _RPT, _RPT)])

    @pl.when(s == 15)
    def _():
        pltpu.sync_copy(accum.at[pl.ds(16 * _RPT, N - 16 * _RPT)],
                        out_hbm.at[c, pl.ds(16 * _RPT, N - 16 * _RPT)])


def kernel(x, edge_index, edge_attr, W, b, root_emb, bond_table):
    row = edge_index[0]
    col = edge_index[1]

    rows_a = jnp.concatenate(
        [row, jnp.full((32 * EA_PAD - E,), PAD_IDX, jnp.int32)]
    ).reshape(32, _A_CHUNKS, 16)

    row_c = row.reshape(16, _C_CHUNKS, 16)
    col_c = col.reshape(16, _C_CHUNKS, 16)
    attr_c = edge_attr.reshape(16, _C_CHUNKS, 16)
    bond2 = jnp.stack([bond_table[:, :HALF], bond_table[:, HALF:]])

    degp = _deg_kernel(rows_a).reshape(2, NPAD, 1)

    h2, base2, dis = _mm_call(x, W, b.reshape(1, D),
                              root_emb.reshape(1, D), degp)
    out2 = _msg_kernel(row_c, col_c, attr_c, dis.reshape(NPAD),
                       bond2, h2, base2)
    return jnp.concatenate([out2[0], out2[1]], axis=1)


# speedup vs baseline: 4.5065x; 3.2180x over previous
import functools

import jax
import jax.numpy as jnp
from jax import lax
from jax.experimental import pallas as pl
from jax.experimental.pallas import tpu as pltpu
from jax.experimental.pallas import tpu_sc as plsc

N = 10000
E = 160000
D = 256
HALF = 128
NPAD = 10240
NROWS = NPAD // 128
PAD_IDX = 10200

EA = E // 32
EA_PAD = 5008
_A_CHUNKS = EA_PAD // 16

_mesh = plsc.VectorSubcoreMesh(core_axis_name="c", subcore_axis_name="s")
_sc_params = pltpu.CompilerParams(needs_layout_passes=False,
                                  use_tc_tiling_on_sc=False)


_SEG = NPAD // 16


@functools.partial(
    pl.kernel,
    out_type=jax.ShapeDtypeStruct((2, NPAD), jnp.float32),
    mesh=_mesh,
    scratch_types=[
        pltpu.VMEM((_A_CHUNKS, 16), jnp.int32),
        pltpu.VMEM((NPAD,), jnp.float32),
        pltpu.VMEM((16, _SEG), jnp.float32),
        pltpu.VMEM((_SEG,), jnp.float32),
        pltpu.VMEM_SHARED((16, NPAD), jnp.float32),
    ],
    compiler_params=_sc_params,
)
def _deg_kernel(rows_hbm, out_hbm, rvm, hist, stage, seg, shared):
    c = lax.axis_index("c")
    s = lax.axis_index("s")
    tid = c * 16 + s
    zeros16 = jnp.zeros((16,), jnp.float32)

    def zrow(j, _):
        hist[pl.ds(j * 16, 16)] = zeros16
        return 0
    lax.fori_loop(0, NPAD // 16, zrow, 0)

    pltpu.sync_copy(rows_hbm.at[tid], rvm)

    ones = jnp.full((16,), 1.0, jnp.float32)

    def chunk(j, _):
        plsc.addupdate_scatter(hist, [rvm[j]], ones)
        return 0
    lax.fori_loop(0, _A_CHUNKS, chunk, 0)

    pltpu.sync_copy(hist, shared.at[s])
    plsc.subcore_barrier()
    pltpu.sync_copy(shared.at[pl.ds(0, 16), pl.ds(s * _SEG, _SEG)], stage)

    def red(k, _):
        v = stage[0, pl.ds(k * 16, 16)]
        for r in range(1, 16):
            v = v + stage[r, pl.ds(k * 16, 16)]
        seg[pl.ds(k * 16, 16)] = v
        return 0
    lax.fori_loop(0, _SEG // 16, red, 0)

    pltpu.sync_copy(seg, out_hbm.at[c, pl.ds(s * _SEG, _SEG)])


BM = 1024
_GRID = (NPAD // BM,)


def _mm_body(x_ref, w_ref, b_ref, root_ref, degp_ref,
             h2_ref, base2_ref, dis_ref):
    h = jnp.dot(x_ref[...], w_ref[...],
                preferred_element_type=jnp.float32) + b_ref[...]
    deg = degp_ref[0] + degp_ref[1] + 1.0
    dis = lax.rsqrt(deg)
    inv = dis * dis
    base = jnp.maximum(h + root_ref[...], 0.0) * inv
    h2_ref[0] = h[:, :HALF]
    h2_ref[1] = h[:, HALF:]
    base2_ref[0] = base[:, :HALF]
    base2_ref[1] = base[:, HALF:]
    dis_ref[...] = dis


_mm_call = pl.pallas_call(
    _mm_body,
    grid=_GRID,
    in_specs=[
        pl.BlockSpec((BM, D), lambda i: (i, 0)),
        pl.BlockSpec((D, D), lambda i: (0, 0)),
        pl.BlockSpec((1, D), lambda i: (0, 0)),
        pl.BlockSpec((1, D), lambda i: (0, 0)),
        pl.BlockSpec((2, BM, 1), lambda i: (0, i, 0)),
    ],
    out_specs=[
        pl.BlockSpec((2, BM, HALF), lambda i: (0, i, 0)),
        pl.BlockSpec((2, BM, HALF), lambda i: (0, i, 0)),
        pl.BlockSpec((BM, 1), lambda i: (i, 0)),
    ],
    out_shape=[
        jax.ShapeDtypeStruct((2, N, HALF), jnp.float32),
        jax.ShapeDtypeStruct((2, N, HALF), jnp.float32),
        jax.ShapeDtypeStruct((NPAD, 1), jnp.float32),
    ],
)


EC = E // 16
CHUNK = 64
_C_CHUNKS = 160
EC_PAD = CHUNK * _C_CHUNKS
NACC = N + (EC_PAD - EC)
_RPT = 624
_SUBS = CHUNK // 16


_GDN = lax.GatherDimensionNumbers(
    offset_dims=(), collapsed_slice_dims=(0,), start_index_map=(0,))


def _vbcast(x, ev):
    return lax.gather(x, ev[:, None], _GDN, (1,),
                      mode=lax.GatherScatterMode.PROMISE_IN_BOUNDS)


@functools.partial(
    pl.kernel,
    out_type=jax.ShapeDtypeStruct((2, N, HALF), jnp.float32),
    mesh=_mesh,
    scratch_types=[
        pltpu.VMEM((NPAD,), jnp.float32),
        pltpu.VMEM((8, HALF), jnp.float32),
        pltpu.VMEM((3, CHUNK), jnp.int32),
        pltpu.VMEM((3, CHUNK), jnp.int32),
        pltpu.VMEM((CHUNK, HALF), jnp.float32),
        pltpu.VMEM((CHUNK, HALF), jnp.float32),
        pltpu.VMEM((CHUNK, HALF), jnp.float32),
        pltpu.VMEM_SHARED((NACC, HALF), jnp.float32),
        pltpu.SemaphoreType.DMA,
        pltpu.SemaphoreType.DMA,
    ],
    compiler_params=_sc_params,
)
def _msg_kernel(ids_hbm, dis_hbm, bond_hbm, h_hbm, base_hbm, out_hbm,
                disv, bondv, ibufa, ibufb, hbufa, hbufb, msg, accum,
                sga, sgb):
    c = lax.axis_index("c")
    s = lax.axis_index("s")
    lanes = lax.iota(jnp.int32, 16)

    pltpu.sync_copy(dis_hbm, disv)
    pltpu.sync_copy(bond_hbm.at[c], bondv)
    pltpu.sync_copy(base_hbm.at[c, pl.ds(s * _RPT, _RPT)],
                    accum.at[pl.ds(s * _RPT, _RPT)])

    @pl.when(s == 15)
    def _():
        pltpu.sync_copy(base_hbm.at[c, pl.ds(16 * _RPT, N - 16 * _RPT)],
                        accum.at[pl.ds(16 * _RPT, N - 16 * _RPT)])
    plsc.subcore_barrier()

    def compute(hbuf, ibuf):
        def sub(t, _):
            base_e = t * 16
            rvec = ibuf[0, pl.ds(base_e, 16)]
            cvec = ibuf[1, pl.ds(base_e, 16)]
            avec = ibuf[2, pl.ds(base_e, 16)]
            norm = (plsc.load_gather(disv, [rvec]) *
                    plsc.load_gather(disv, [cvec]))
            for e in range(16):
                ev = jnp.full((16,), e, jnp.int32)
                ne = _vbcast(norm, ev)
                ab = _vbcast(avec, ev)
                for g in range(8):
                    hseg = hbuf[base_e + e, pl.ds(g * 16, 16)]
                    bseg = plsc.load_gather(bondv, [ab, lanes + g * 16])
                    msg[base_e + e, pl.ds(g * 16, 16)] = (
                        jnp.maximum(hseg + bseg, 0.0) * ne)
            return 0
        lax.fori_loop(0, _SUBS, sub, 0)

    def pair(jj, _):
        a = jj * 2
        pltpu.sync_copy(ids_hbm.at[s, a], ibufa)
        dga = pltpu.async_copy(h_hbm.at[c].at[ibufa.at[0]], hbufa, sga)
        pltpu.sync_copy(ids_hbm.at[s, a + 1], ibufb)
        dgb = pltpu.async_copy(h_hbm.at[c].at[ibufb.at[0]], hbufb, sgb)
        dga.wait()
        compute(hbufa, ibufa)
        pltpu.sync_copy(msg, accum.at[ibufa.at[1]], add=True)
        dgb.wait()
        compute(hbufb, ibufb)
        pltpu.sync_copy(msg, accum.at[ibufb.at[1]], add=True)
        return 0
    lax.fori_loop(0, _C_CHUNKS // 2, pair, 0)

    plsc.subcore_barrier()
    pltpu.sync_copy(accum.at[pl.ds(s * _RPT, _RPT)],
                    out_hbm.at[c, pl.ds(s * _RPT, _RPT)])

    @pl.when(s == 15)
    def _():
        pltpu.sync_copy(accum.at[pl.ds(16 * _RPT, N - 16 * _RPT)],
                        out_hbm.at[c, pl.ds(16 * _RPT, N - 16 * _RPT)])


def kernel(x, edge_index, edge_attr, W, b, root_emb, bond_table):
    row = edge_index[0]
    col = edge_index[1]

    rows_a = jnp.concatenate(
        [row, jnp.full((32 * EA_PAD - E,), PAD_IDX, jnp.int32)]
    ).reshape(32, _A_CHUNKS, 16)

    def _pad_c(a, fill):
        pad = jnp.full((16, EC_PAD - EC), fill, jnp.int32)
        return jnp.concatenate([a.reshape(16, EC), pad],
                               axis=1).reshape(16, _C_CHUNKS, 1, CHUNK)

    ids_c = jnp.concatenate(
        [_pad_c(row, 0), _pad_c(col, N), _pad_c(edge_attr, 0)], axis=2)
    bond2 = jnp.stack([bond_table[:, :HALF], bond_table[:, HALF:]])

    degp = _deg_kernel(rows_a).reshape(2, NPAD, 1)

    h2, base2, dis = _mm_call(x, W, b.reshape(1, D),
                              root_emb.reshape(1, D), degp)
    out2 = _msg_kernel(ids_c, dis.reshape(NPAD), bond2, h2, base2)
    return jnp.concatenate([out2[0], out2[1]], axis=1)


# speedup vs baseline: 5.1294x; 1.1382x over previous
import functools

import jax
import jax.numpy as jnp
from jax import lax
from jax.experimental import pallas as pl
from jax.experimental.pallas import tpu as pltpu
from jax.experimental.pallas import tpu_sc as plsc

N = 10000
E = 160000
D = 256
HALF = 128
NPAD = 10240
NROWS = NPAD // 128
PAD_IDX = 10200

EA = E // 32
EA_PAD = 5008
_A_CHUNKS = EA_PAD // 16

_mesh = plsc.VectorSubcoreMesh(core_axis_name="c", subcore_axis_name="s")
_sc_params = pltpu.CompilerParams(needs_layout_passes=False,
                                  use_tc_tiling_on_sc=False)


_SEG = NPAD // 16


@functools.partial(
    pl.kernel,
    out_type=jax.ShapeDtypeStruct((2, NPAD), jnp.float32),
    mesh=_mesh,
    scratch_types=[
        pltpu.VMEM((_A_CHUNKS, 16), jnp.int32),
        pltpu.VMEM((NPAD,), jnp.float32),
        pltpu.VMEM((16, _SEG), jnp.float32),
        pltpu.VMEM((_SEG,), jnp.float32),
        pltpu.VMEM_SHARED((16, NPAD), jnp.float32),
    ],
    compiler_params=_sc_params,
)
def _deg_kernel(rows_hbm, out_hbm, rvm, hist, stage, seg, shared):
    c = lax.axis_index("c")
    s = lax.axis_index("s")
    tid = c * 16 + s
    zeros16 = jnp.zeros((16,), jnp.float32)

    def zrow(j, _):
        hist[pl.ds(j * 16, 16)] = zeros16
        return 0
    lax.fori_loop(0, NPAD // 16, zrow, 0)

    pltpu.sync_copy(rows_hbm.at[tid], rvm)

    ones = jnp.full((16,), 1.0, jnp.float32)

    def chunk(j, _):
        plsc.addupdate_scatter(hist, [rvm[j]], ones)
        return 0
    lax.fori_loop(0, _A_CHUNKS, chunk, 0)

    pltpu.sync_copy(hist, shared.at[s])
    plsc.subcore_barrier()
    pltpu.sync_copy(shared.at[pl.ds(0, 16), pl.ds(s * _SEG, _SEG)], stage)

    def red(k, _):
        v = stage[0, pl.ds(k * 16, 16)]
        for r in range(1, 16):
            v = v + stage[r, pl.ds(k * 16, 16)]
        seg[pl.ds(k * 16, 16)] = v
        return 0
    lax.fori_loop(0, _SEG // 16, red, 0)

    pltpu.sync_copy(seg, out_hbm.at[c, pl.ds(s * _SEG, _SEG)])


BM = 1024
_GRID = (NPAD // BM,)


def _mm_body(x_ref, w_ref, b_ref, root_ref, degp_ref,
             h2_ref, base2_ref, dis_ref):
    h = jnp.dot(x_ref[...], w_ref[...],
                preferred_element_type=jnp.float32) + b_ref[...]
    deg = degp_ref[0] + degp_ref[1] + 1.0
    dis = lax.rsqrt(deg)
    inv = dis * dis
    base = jnp.maximum(h + root_ref[...], 0.0) * inv
    h2_ref[0] = h[:, :HALF]
    h2_ref[1] = h[:, HALF:]
    base2_ref[0] = base[:, :HALF]
    base2_ref[1] = base[:, HALF:]
    dis_ref[...] = dis


_mm_call = pl.pallas_call(
    _mm_body,
    grid=_GRID,
    in_specs=[
        pl.BlockSpec((BM, D), lambda i: (i, 0)),
        pl.BlockSpec((D, D), lambda i: (0, 0)),
        pl.BlockSpec((1, D), lambda i: (0, 0)),
        pl.BlockSpec((1, D), lambda i: (0, 0)),
        pl.BlockSpec((2, BM, 1), lambda i: (0, i, 0)),
    ],
    out_specs=[
        pl.BlockSpec((2, BM, HALF), lambda i: (0, i, 0)),
        pl.BlockSpec((2, BM, HALF), lambda i: (0, i, 0)),
        pl.BlockSpec((BM, 1), lambda i: (i, 0)),
    ],
    out_shape=[
        jax.ShapeDtypeStruct((2, N, HALF), jnp.float32),
        jax.ShapeDtypeStruct((2, N, HALF), jnp.float32),
        jax.ShapeDtypeStruct((NPAD, 1), jnp.float32),
    ],
)


EC = E // 16
CHUNK = 64
_C_CHUNKS = 160
EC_PAD = CHUNK * _C_CHUNKS
NACC = N + (EC_PAD - EC)
_RPT = 624
_SUBS = CHUNK // 16


_GDN = lax.GatherDimensionNumbers(
    offset_dims=(), collapsed_slice_dims=(0,), start_index_map=(0,))


def _vbcast(x, ev):
    return lax.gather(x, ev[:, None], _GDN, (1,),
                      mode=lax.GatherScatterMode.PROMISE_IN_BOUNDS)


GRP = 8
_GROUPS = _C_CHUNKS // GRP


@functools.partial(
    pl.kernel,
    out_type=jax.ShapeDtypeStruct((2, N, HALF), jnp.float32),
    mesh=_mesh,
    scratch_types=[
        pltpu.VMEM((NPAD,), jnp.float32),
        pltpu.VMEM((8, HALF), jnp.float32),
        pltpu.VMEM((GRP, 3, CHUNK), jnp.int32),
        pltpu.VMEM((CHUNK, HALF), jnp.float32),
        pltpu.VMEM((CHUNK, HALF), jnp.float32),
        pltpu.VMEM((CHUNK, HALF), jnp.float32),
        pltpu.VMEM((CHUNK, HALF), jnp.float32),
        pltpu.VMEM_SHARED((NACC, HALF), jnp.float32),
        pltpu.SemaphoreType.DMA,
        pltpu.SemaphoreType.DMA,
        pltpu.SemaphoreType.DMA,
        pltpu.SemaphoreType.DMA,
    ],
    compiler_params=_sc_params,
)
def _msg_kernel(ids_hbm, dis_hbm, bond_hbm, h_hbm, base_hbm, out_hbm,
                disv, bondv, ibig, hbufa, hbufb, msga, msgb, accum,
                sga, sgb, ssa, ssb):
    c = lax.axis_index("c")
    s = lax.axis_index("s")
    lanes = lax.iota(jnp.int32, 16)

    pltpu.sync_copy(dis_hbm, disv)
    pltpu.sync_copy(bond_hbm.at[c], bondv)
    pltpu.sync_copy(base_hbm.at[c, pl.ds(s * _RPT, _RPT)],
                    accum.at[pl.ds(s * _RPT, _RPT)])

    @pl.when(s == 15)
    def _():
        pltpu.sync_copy(base_hbm.at[c, pl.ds(16 * _RPT, N - 16 * _RPT)],
                        accum.at[pl.ds(16 * _RPT, N - 16 * _RPT)])
    plsc.subcore_barrier()

    def compute(hbuf, msg, k):
        def sub(t, _):
            base_e = t * 16
            rvec = ibig[k, 0, pl.ds(base_e, 16)]
            cvec = ibig[k, 1, pl.ds(base_e, 16)]
            avec = ibig[k, 2, pl.ds(base_e, 16)]
            norm = (plsc.load_gather(disv, [rvec]) *
                    plsc.load_gather(disv, [cvec]))
            for e in range(16):
                ev = jnp.full((16,), e, jnp.int32)
                ne = _vbcast(norm, ev)
                ab = _vbcast(avec, ev)
                for g in range(8):
                    hseg = hbuf[base_e + e, pl.ds(g * 16, 16)]
                    bseg = plsc.load_gather(bondv, [ab, lanes + g * 16])
                    msg[base_e + e, pl.ds(g * 16, 16)] = (
                        jnp.maximum(hseg + bseg, 0.0) * ne)
            return 0
        lax.fori_loop(0, _SUBS, sub, 0)

    hb = (hbufa, hbufb)
    mb = (msga, msgb)
    gsem = (sga, sgb)
    ssem = (ssa, ssb)

    def group(g, _):
        pltpu.sync_copy(ids_hbm.at[s, g], ibig)
        gd = [pltpu.async_copy(h_hbm.at[c].at[ibig.at[k, 0]], hb[k], gsem[k])
              for k in range(2)]
        sd = [None, None]
        for k in range(GRP):
            p = k & 1
            if sd[p] is not None:
                sd[p].wait()
            gd[p].wait()
            compute(hb[p], mb[p], k)
            sd[p] = pltpu.async_copy(mb[p], accum.at[ibig.at[k, 1]],
                                     ssem[p], add=True)
            if k + 2 < GRP:
                gd[p] = pltpu.async_copy(h_hbm.at[c].at[ibig.at[k + 2, 0]],
                                         hb[p], gsem[p])
        sd[0].wait()
        sd[1].wait()
        return 0
    lax.fori_loop(0, _GROUPS, group, 0)

    plsc.subcore_barrier()
    pltpu.sync_copy(accum.at[pl.ds(s * _RPT, _RPT)],
                    out_hbm.at[c, pl.ds(s * _RPT, _RPT)])

    @pl.when(s == 15)
    def _():
        pltpu.sync_copy(accum.at[pl.ds(16 * _RPT, N - 16 * _RPT)],
                        out_hbm.at[c, pl.ds(16 * _RPT, N - 16 * _RPT)])


def kernel(x, edge_index, edge_attr, W, b, root_emb, bond_table):
    row = edge_index[0]
    col = edge_index[1]

    rows_a = jnp.concatenate(
        [row, jnp.full((32 * EA_PAD - E,), PAD_IDX, jnp.int32)]
    ).reshape(32, _A_CHUNKS, 16)

    def _pad_c(a, fill):
        pad = jnp.full((16, EC_PAD - EC), fill, jnp.int32)
        return jnp.concatenate([a.reshape(16, EC), pad],
                               axis=1).reshape(16, _C_CHUNKS, 1, CHUNK)

    ids_c = jnp.concatenate(
        [_pad_c(row, 0), _pad_c(col, N), _pad_c(edge_attr, 0)],
        axis=2).reshape(16, _GROUPS, GRP, 3, CHUNK)
    bond2 = jnp.stack([bond_table[:, :HALF], bond_table[:, HALF:]])

    degp = _deg_kernel(rows_a).reshape(2, NPAD, 1)

    h2, base2, dis = _mm_call(x, W, b.reshape(1, D),
                              root_emb.reshape(1, D), degp)
    out2 = _msg_kernel(ids_c, dis.reshape(NPAD), bond2, h2, base2)
    return jnp.concatenate([out2[0], out2[1]], axis=1)


# speedup vs baseline: 7.3324x; 1.4295x over previous
import functools

import jax
import jax.numpy as jnp
from jax import lax
from jax.experimental import pallas as pl
from jax.experimental.pallas import tpu as pltpu
from jax.experimental.pallas import tpu_sc as plsc

N = 10000
E = 160000
D = 256
HALF = 128
NPAD = 10240
NROWS = NPAD // 128
PAD_IDX = 10200

EA = E // 32
EA_PAD = 5008
_A_CHUNKS = EA_PAD // 16

_mesh = plsc.VectorSubcoreMesh(core_axis_name="c", subcore_axis_name="s")
_sc_params = pltpu.CompilerParams(needs_layout_passes=False,
                                  use_tc_tiling_on_sc=False)


_SEG = NPAD // 16


@functools.partial(
    pl.kernel,
    out_type=jax.ShapeDtypeStruct((2, NPAD), jnp.float32),
    mesh=_mesh,
    scratch_types=[
        pltpu.VMEM((_A_CHUNKS, 16), jnp.int32),
        pltpu.VMEM((NPAD,), jnp.float32),
        pltpu.VMEM((16, _SEG), jnp.float32),
        pltpu.VMEM((_SEG,), jnp.float32),
        pltpu.VMEM_SHARED((16, NPAD), jnp.float32),
    ],
    compiler_params=_sc_params,
)
def _deg_kernel(rows_hbm, out_hbm, rvm, hist, stage, seg, shared):
    c = lax.axis_index("c")
    s = lax.axis_index("s")
    tid = c * 16 + s
    zeros16 = jnp.zeros((16,), jnp.float32)

    def zrow(j, _):
        hist[pl.ds(j * 16, 16)] = zeros16
        return 0
    lax.fori_loop(0, NPAD // 16, zrow, 0)

    pltpu.sync_copy(rows_hbm.at[tid], rvm)

    ones = jnp.full((16,), 1.0, jnp.float32)

    def chunk(j, _):
        plsc.addupdate_scatter(hist, [rvm[j]], ones)
        return 0
    lax.fori_loop(0, _A_CHUNKS, chunk, 0)

    pltpu.sync_copy(hist, shared.at[s])
    plsc.subcore_barrier()
    pltpu.sync_copy(shared.at[pl.ds(0, 16), pl.ds(s * _SEG, _SEG)], stage)

    def red(k, _):
        v = stage[0, pl.ds(k * 16, 16)]
        for r in range(1, 16):
            v = v + stage[r, pl.ds(k * 16, 16)]
        seg[pl.ds(k * 16, 16)] = v
        return 0
    lax.fori_loop(0, _SEG // 16, red, 0)

    pltpu.sync_copy(seg, out_hbm.at[c, pl.ds(s * _SEG, _SEG)])


BM = 1024
_GRID = (NPAD // BM,)


def _mm_body(x_ref, w_ref, b_ref, root_ref, degp_ref,
             h2_ref, base2_ref, dis_ref):
    h = jnp.dot(x_ref[...], w_ref[...],
                preferred_element_type=jnp.float32) + b_ref[...]
    deg = degp_ref[0] + degp_ref[1] + 1.0
    dis = lax.rsqrt(deg)
    inv = dis * dis
    base = jnp.maximum(h + root_ref[...], 0.0) * inv
    h2_ref[0] = h[:, :HALF].astype(jnp.bfloat16)
    h2_ref[1] = h[:, HALF:].astype(jnp.bfloat16)
    base2_ref[0] = base[:, :HALF]
    base2_ref[1] = base[:, HALF:]
    dis_ref[...] = dis


_mm_call = pl.pallas_call(
    _mm_body,
    grid=_GRID,
    in_specs=[
        pl.BlockSpec((BM, D), lambda i: (i, 0)),
        pl.BlockSpec((D, D), lambda i: (0, 0)),
        pl.BlockSpec((1, D), lambda i: (0, 0)),
        pl.BlockSpec((1, D), lambda i: (0, 0)),
        pl.BlockSpec((2, BM, 1), lambda i: (0, i, 0)),
    ],
    out_specs=[
        pl.BlockSpec((2, BM, HALF), lambda i: (0, i, 0)),
        pl.BlockSpec((2, BM, HALF), lambda i: (0, i, 0)),
        pl.BlockSpec((BM, 1), lambda i: (i, 0)),
    ],
    out_shape=[
        jax.ShapeDtypeStruct((2, N, HALF), jnp.bfloat16),
        jax.ShapeDtypeStruct((2, N, HALF), jnp.float32),
        jax.ShapeDtypeStruct((NPAD, 1), jnp.float32),
    ],
)


EC = E // 16
CHUNK = 64
_C_CHUNKS = 160
EC_PAD = CHUNK * _C_CHUNKS
NACC = N + (EC_PAD - EC)
_RPT = 624
_SUBS = CHUNK // 16


_GDN = lax.GatherDimensionNumbers(
    offset_dims=(), collapsed_slice_dims=(0,), start_index_map=(0,))


def _vbcast(x, ev):
    return lax.gather(x, ev[:, None], _GDN, (1,),
                      mode=lax.GatherScatterMode.PROMISE_IN_BOUNDS)


GRP = 8
_GROUPS = _C_CHUNKS // GRP


@functools.partial(
    pl.kernel,
    out_type=jax.ShapeDtypeStruct((2, N, HALF), jnp.float32),
    mesh=_mesh,
    scratch_types=[
        pltpu.VMEM((NPAD,), jnp.float32),
        pltpu.VMEM((8, HALF), jnp.float32),
        pltpu.VMEM((GRP, 3, CHUNK), jnp.int32),
        pltpu.VMEM((CHUNK, HALF), jnp.bfloat16),
        pltpu.VMEM((CHUNK, HALF), jnp.bfloat16),
        pltpu.VMEM((CHUNK, HALF), jnp.float32),
        pltpu.VMEM((CHUNK, HALF), jnp.float32),
        pltpu.VMEM_SHARED((NACC, HALF), jnp.float32),
        pltpu.SemaphoreType.DMA,
        pltpu.SemaphoreType.DMA,
        pltpu.SemaphoreType.DMA,
        pltpu.SemaphoreType.DMA,
    ],
    compiler_params=_sc_params,
)
def _msg_kernel(ids_hbm, dis_hbm, bond_hbm, h_hbm, base_hbm, out_hbm,
                disv, bondv, ibig, hbufa, hbufb, msga, msgb, accum,
                sga, sgb, ssa, ssb):
    c = lax.axis_index("c")
    s = lax.axis_index("s")
    lanes = lax.iota(jnp.int32, 16)

    pltpu.sync_copy(dis_hbm, disv)
    pltpu.sync_copy(bond_hbm.at[c], bondv)
    pltpu.sync_copy(base_hbm.at[c, pl.ds(s * _RPT, _RPT)],
                    accum.at[pl.ds(s * _RPT, _RPT)])

    @pl.when(s == 15)
    def _():
        pltpu.sync_copy(base_hbm.at[c, pl.ds(16 * _RPT, N - 16 * _RPT)],
                        accum.at[pl.ds(16 * _RPT, N - 16 * _RPT)])
    plsc.subcore_barrier()

    def compute(hbuf, msg, k):
        def sub(t, _):
            base_e = t * 16
            rvec = ibig[k, 0, pl.ds(base_e, 16)]
            cvec = ibig[k, 1, pl.ds(base_e, 16)]
            avec = ibig[k, 2, pl.ds(base_e, 16)]
            norm = (plsc.load_gather(disv, [rvec]) *
                    plsc.load_gather(disv, [cvec]))
            for e in range(16):
                ev = jnp.full((16,), e, jnp.int32)
                ne = _vbcast(norm, ev)
                ab = _vbcast(avec, ev)
                rsplat = jnp.full((16,), base_e + e, jnp.int32)
                for g in range(4):
                    hw = plsc.bitcast(hbuf[base_e + e, pl.ds(g * 32, 32)],
                                      jnp.int32)
                    hlo = plsc.bitcast(lax.shift_left(hw, 16), jnp.float32)
                    hhi = plsc.bitcast(
                        lax.bitwise_and(hw, jnp.int32(-65536)), jnp.float32)
                    ceven = lanes * 2 + g * 32
                    blo = plsc.load_gather(bondv, [ab, ceven])
                    bhi = plsc.load_gather(bondv, [ab, ceven + 1])
                    plsc.store_scatter(
                        msg, [rsplat, ceven],
                        jnp.maximum(hlo + blo, 0.0) * ne)
                    plsc.store_scatter(
                        msg, [rsplat, ceven + 1],
                        jnp.maximum(hhi + bhi, 0.0) * ne)
            return 0
        lax.fori_loop(0, _SUBS, sub, 0)

    hb = (hbufa, hbufb)
    mb = (msga, msgb)
    gsem = (sga, sgb)
    ssem = (ssa, ssb)

    def group(g, _):
        pltpu.sync_copy(ids_hbm.at[s, g], ibig)
        gd = [pltpu.async_copy(h_hbm.at[c].at[ibig.at[k, 0]], hb[k], gsem[k])
              for k in range(2)]
        sd = [None, None]
        for k in range(GRP):
            p = k & 1
            if sd[p] is not None:
                sd[p].wait()
            gd[p].wait()
            compute(hb[p], mb[p], k)
            sd[p] = pltpu.async_copy(mb[p], accum.at[ibig.at[k, 1]],
                                     ssem[p], add=True)
            if k + 2 < GRP:
                gd[p] = pltpu.async_copy(h_hbm.at[c].at[ibig.at[k + 2, 0]],
                                         hb[p], gsem[p])
        sd[0].wait()
        sd[1].wait()
        return 0
    lax.fori_loop(0, _GROUPS, group, 0)

    plsc.subcore_barrier()
    pltpu.sync_copy(accum.at[pl.ds(s * _RPT, _RPT)],
                    out_hbm.at[c, pl.ds(s * _RPT, _RPT)])

    @pl.when(s == 15)
    def _():
        pltpu.sync_copy(accum.at[pl.ds(16 * _RPT, N - 16 * _RPT)],
                        out_hbm.at[c, pl.ds(16 * _RPT, N - 16 * _RPT)])


def kernel(x, edge_index, edge_attr, W, b, root_emb, bond_table):
    row = edge_index[0]
    col = edge_index[1]

    rows_a = jnp.concatenate(
        [row, jnp.full((32 * EA_PAD - E,), PAD_IDX, jnp.int32)]
    ).reshape(32, _A_CHUNKS, 16)

    def _pad_c(a, fill):
        pad = jnp.full((16, EC_PAD - EC), fill, jnp.int32)
        return jnp.concatenate([a.reshape(16, EC), pad],
                               axis=1).reshape(16, _C_CHUNKS, 1, CHUNK)

    ids_c = jnp.concatenate(
        [_pad_c(row, 0), _pad_c(col, N), _pad_c(edge_attr, 0)],
        axis=2).reshape(16, _GROUPS, GRP, 3, CHUNK)
    bond2 = jnp.stack([bond_table[:, :HALF], bond_table[:, HALF:]])

    degp = _deg_kernel(rows_a).reshape(2, NPAD, 1)

    h2, base2, dis = _mm_call(x, W, b.reshape(1, D),
                              root_emb.reshape(1, D), degp)
    out2 = _msg_kernel(ids_c, dis.reshape(NPAD), bond2, h2, base2)
    return jnp.concatenate([out2[0], out2[1]], axis=1)


# speedup vs baseline: 7.3712x; 1.0053x over previous
import functools

import jax
import jax.numpy as jnp
from jax import lax
from jax.experimental import pallas as pl
from jax.experimental.pallas import tpu as pltpu
from jax.experimental.pallas import tpu_sc as plsc

N = 10000
E = 160000
D = 256
HALF = 128
NPAD = 10240
NROWS = NPAD // 128
PAD_IDX = 10200

EA = E // 32
EA_PAD = 5008
_A_CHUNKS = EA_PAD // 16

_mesh = plsc.VectorSubcoreMesh(core_axis_name="c", subcore_axis_name="s")
_sc_params = pltpu.CompilerParams(needs_layout_passes=False,
                                  use_tc_tiling_on_sc=False)


_SEG = NPAD // 16


@functools.partial(
    pl.kernel,
    out_type=jax.ShapeDtypeStruct((2, NPAD), jnp.float32),
    mesh=_mesh,
    scratch_types=[
        pltpu.VMEM((_A_CHUNKS, 16), jnp.int32),
        pltpu.VMEM((NPAD,), jnp.float32),
        pltpu.VMEM((16, _SEG), jnp.float32),
        pltpu.VMEM((_SEG,), jnp.float32),
        pltpu.VMEM_SHARED((16, NPAD), jnp.float32),
    ],
    compiler_params=_sc_params,
)
def _deg_kernel(rows_hbm, out_hbm, rvm, hist, stage, seg, shared):
    c = lax.axis_index("c")
    s = lax.axis_index("s")
    tid = c * 16 + s
    zeros16 = jnp.zeros((16,), jnp.float32)

    def zrow(j, _):
        hist[pl.ds(j * 16, 16)] = zeros16
        return 0
    lax.fori_loop(0, NPAD // 16, zrow, 0)

    pltpu.sync_copy(rows_hbm.at[tid], rvm)

    ones = jnp.full((16,), 1.0, jnp.float32)

    def chunk(j, _):
        plsc.addupdate_scatter(hist, [rvm[j]], ones)
        return 0
    lax.fori_loop(0, _A_CHUNKS, chunk, 0)

    pltpu.sync_copy(hist, shared.at[s])
    plsc.subcore_barrier()
    pltpu.sync_copy(shared.at[pl.ds(0, 16), pl.ds(s * _SEG, _SEG)], stage)

    def red(k, _):
        v = stage[0, pl.ds(k * 16, 16)]
        for r in range(1, 16):
            v = v + stage[r, pl.ds(k * 16, 16)]
        seg[pl.ds(k * 16, 16)] = v
        return 0
    lax.fori_loop(0, _SEG // 16, red, 0)

    pltpu.sync_copy(seg, out_hbm.at[c, pl.ds(s * _SEG, _SEG)])


BM = 1024
_GRID = (NPAD // BM,)


def _mm_body(x_ref, w_ref, b_ref, root_ref, degp_ref,
             h2_ref, base2_ref, dis_ref):
    h = jnp.dot(x_ref[...], w_ref[...],
                preferred_element_type=jnp.float32) + b_ref[...]
    deg = degp_ref[0] + degp_ref[1] + 1.0
    dis = lax.rsqrt(deg)
    inv = dis * dis
    base = jnp.maximum(h + root_ref[...], 0.0) * inv
    h2_ref[0] = h[:, :HALF].astype(jnp.bfloat16)
    h2_ref[1] = h[:, HALF:].astype(jnp.bfloat16)
    base2_ref[0] = base[:, :HALF]
    base2_ref[1] = base[:, HALF:]
    dis_ref[...] = dis


_mm_call = pl.pallas_call(
    _mm_body,
    grid=_GRID,
    in_specs=[
        pl.BlockSpec((BM, D), lambda i: (i, 0)),
        pl.BlockSpec((D, D), lambda i: (0, 0)),
        pl.BlockSpec((1, D), lambda i: (0, 0)),
        pl.BlockSpec((1, D), lambda i: (0, 0)),
        pl.BlockSpec((2, BM, 1), lambda i: (0, i, 0)),
    ],
    out_specs=[
        pl.BlockSpec((2, BM, HALF), lambda i: (0, i, 0)),
        pl.BlockSpec((2, BM, HALF), lambda i: (0, i, 0)),
        pl.BlockSpec((BM, 1), lambda i: (i, 0)),
    ],
    out_shape=[
        jax.ShapeDtypeStruct((2, N, HALF), jnp.bfloat16),
        jax.ShapeDtypeStruct((2, N, HALF), jnp.float32),
        jax.ShapeDtypeStruct((NPAD, 1), jnp.float32),
    ],
)


EC = E // 16
CHUNK = 64
_C_CHUNKS = 160
EC_PAD = CHUNK * _C_CHUNKS
NACC = N + (EC_PAD - EC)
_RPT = 624
_SUBS = CHUNK // 16


_GDN = lax.GatherDimensionNumbers(
    offset_dims=(), collapsed_slice_dims=(0,), start_index_map=(0,))


def _vbcast(x, ev):
    return lax.gather(x, ev[:, None], _GDN, (1,),
                      mode=lax.GatherScatterMode.PROMISE_IN_BOUNDS)


GRP = 10
_GROUPS = _C_CHUNKS // GRP
_GDEPTH = 3


@functools.partial(
    pl.kernel,
    out_type=jax.ShapeDtypeStruct((2, N, HALF), jnp.float32),
    mesh=_mesh,
    scratch_types=[
        pltpu.VMEM((NPAD,), jnp.float32),
        pltpu.VMEM((8, HALF), jnp.float32),
        pltpu.VMEM((GRP, 3, CHUNK), jnp.int32),
        pltpu.VMEM((CHUNK, HALF), jnp.bfloat16),
        pltpu.VMEM((CHUNK, HALF), jnp.bfloat16),
        pltpu.VMEM((CHUNK, HALF), jnp.bfloat16),
        pltpu.VMEM((CHUNK, HALF), jnp.float32),
        pltpu.VMEM((CHUNK, HALF), jnp.float32),
        pltpu.VMEM_SHARED((NACC, HALF), jnp.float32),
        pltpu.SemaphoreType.DMA,
        pltpu.SemaphoreType.DMA,
        pltpu.SemaphoreType.DMA,
        pltpu.SemaphoreType.DMA,
        pltpu.SemaphoreType.DMA,
    ],
    compiler_params=_sc_params,
)
def _msg_kernel(ids_hbm, dis_hbm, bond_hbm, h_hbm, base_hbm, out_hbm,
                disv, bondv, ibig, hbufa, hbufb, hbufc, msga, msgb, accum,
                sga, sgb, sgc, ssa, ssb):
    c = lax.axis_index("c")
    s = lax.axis_index("s")
    lanes = lax.iota(jnp.int32, 16)

    pltpu.sync_copy(dis_hbm, disv)
    pltpu.sync_copy(bond_hbm.at[c], bondv)
    pltpu.sync_copy(base_hbm.at[c, pl.ds(s * _RPT, _RPT)],
                    accum.at[pl.ds(s * _RPT, _RPT)])

    @pl.when(s == 15)
    def _():
        pltpu.sync_copy(base_hbm.at[c, pl.ds(16 * _RPT, N - 16 * _RPT)],
                        accum.at[pl.ds(16 * _RPT, N - 16 * _RPT)])
    plsc.subcore_barrier()

    def compute(hbuf, msg, k):
        def sub(t, _):
            base_e = t * 16
            rvec = ibig[k, 0, pl.ds(base_e, 16)]
            cvec = ibig[k, 1, pl.ds(base_e, 16)]
            avec = ibig[k, 2, pl.ds(base_e, 16)]
            norm = (plsc.load_gather(disv, [rvec]) *
                    plsc.load_gather(disv, [cvec]))
            for e in range(16):
                ev = jnp.full((16,), e, jnp.int32)
                ne = _vbcast(norm, ev)
                ab = _vbcast(avec, ev)
                rsplat = jnp.full((16,), base_e + e, jnp.int32)
                for g in range(4):
                    hw = plsc.bitcast(hbuf[base_e + e, pl.ds(g * 32, 32)],
                                      jnp.int32)
                    hlo = plsc.bitcast(lax.shift_left(hw, 16), jnp.float32)
                    hhi = plsc.bitcast(
                        lax.bitwise_and(hw, jnp.int32(-65536)), jnp.float32)
                    ceven = lanes * 2 + g * 32
                    blo = plsc.load_gather(bondv, [ab, ceven])
                    bhi = plsc.load_gather(bondv, [ab, ceven + 1])
                    plsc.store_scatter(
                        msg, [rsplat, ceven],
                        jnp.maximum(hlo + blo, 0.0) * ne)
                    plsc.store_scatter(
                        msg, [rsplat, ceven + 1],
                        jnp.maximum(hhi + bhi, 0.0) * ne)
            return 0
        lax.fori_loop(0, _SUBS, sub, 0)

    hb = (hbufa, hbufb, hbufc)
    mb = (msga, msgb)
    gsem = (sga, sgb, sgc)
    ssem = (ssa, ssb)

    def group(g, _):
        pltpu.sync_copy(ids_hbm.at[s, g], ibig)
        gd = [pltpu.async_copy(h_hbm.at[c].at[ibig.at[k, 0]], hb[k], gsem[k])
              for k in range(_GDEPTH)]
        sd = [None, None]
        for k in range(GRP):
            p = k % _GDEPTH
            q = k & 1
            if sd[q] is not None:
                sd[q].wait()
            gd[p].wait()
            compute(hb[p], mb[q], k)
            sd[q] = pltpu.async_copy(mb[q], accum.at[ibig.at[k, 1]],
                                     ssem[q], add=True)
            if k + _GDEPTH < GRP:
                gd[p] = pltpu.async_copy(
                    h_hbm.at[c].at[ibig.at[k + _GDEPTH, 0]], hb[p], gsem[p])
        sd[0].wait()
        sd[1].wait()
        return 0
    lax.fori_loop(0, _GROUPS, group, 0)

    plsc.subcore_barrier()
    pltpu.sync_copy(accum.at[pl.ds(s * _RPT, _RPT)],
                    out_hbm.at[c, pl.ds(s * _RPT, _RPT)])

    @pl.when(s == 15)
    def _():
        pltpu.sync_copy(accum.at[pl.ds(16 * _RPT, N - 16 * _RPT)],
                        out_hbm.at[c, pl.ds(16 * _RPT, N - 16 * _RPT)])


def kernel(x, edge_index, edge_attr, W, b, root_emb, bond_table):
    row = edge_index[0]
    col = edge_index[1]

    rows_a = jnp.concatenate(
        [row, jnp.full((32 * EA_PAD - E,), PAD_IDX, jnp.int32)]
    ).reshape(32, _A_CHUNKS, 16)

    def _pad_c(a, fill):
        pad = jnp.full((16, EC_PAD - EC), fill, jnp.int32)
        return jnp.concatenate([a.reshape(16, EC), pad],
                               axis=1).reshape(16, _C_CHUNKS, 1, CHUNK)

    ids_c = jnp.concatenate(
        [_pad_c(row, 0), _pad_c(col, N), _pad_c(edge_attr, 0)],
        axis=2).reshape(16, _GROUPS, GRP, 3, CHUNK)
    bond2 = jnp.stack([bond_table[:, :HALF], bond_table[:, HALF:]])

    degp = _deg_kernel(rows_a).reshape(2, NPAD, 1)

    h2, base2, dis = _mm_call(x, W, b.reshape(1, D),
                              root_emb.reshape(1, D), degp)
    out2 = _msg_kernel(ids_c, dis.reshape(NPAD), bond2, h2, base2)
    return jnp.concatenate([out2[0], out2[1]], axis=1)
